# baseline (speedup 1.0000x reference)
import jax
import jax.numpy as jnp
from jax import lax
from jax.experimental import pallas as pl
from jax.experimental.pallas import tpu as pltpu
from jax.experimental.pallas import tpu_sc as plsc

N = 10000
E = 320000
R = 8
D_IN = 128
D_HID = 256
D_OUT = 128

L = 16
NTILES = 16
NW = 32
E_PAD = 327680
EPW = E_PAD // NW
SEG = 128
ROWS_E = E_PAD // SEG
RPW = EPW // SEG
BROWS = 10
NGRP = BROWS * SEG // L
NBLK = RPW // BROWS
FSEG = 128
RING = 2 * FSEG
CAP = EPW
RN = R * N
CNT_ROWS = 81920
ACC_N = 10240
DUMP_N = N
WB = N // NTILES
ZR = 32

_mesh = plsc.VectorSubcoreMesh(core_axis_name="c", subcore_axis_name="s")
_params = pltpu.CompilerParams(use_tc_tiling_on_sc=False,
                               needs_layout_passes=False)


def _partition():
  def body(srcm, dstm, zcnt, lsrc, lrow, nseg_o, cnt_a, cnt_b, cnt_acc,
           sbuf, dbuf, ring_s, ring_r, nsvb, ones_v, sem_c, *sem_f):
    cid = lax.axis_index("c")
    sid = lax.axis_index("s")
    w = cid * NTILES + sid
    it16 = lax.iota(jnp.int32, 16)

    pltpu.sync_copy(zcnt, cnt_acc.at[pl.ds(sid * (CNT_ROWS // NTILES),
                                           CNT_ROWS // NTILES)])

    def fill_ones(i, c):
      ones_v[pl.ds(i * L, L)] = jnp.ones((L,), jnp.float32)
      return c
    lax.fori_loop(0, SEG // L, fill_ones, 0)
    plsc.subcore_barrier()

    zero = jnp.int32(0)
    state = tuple([zero] * 8 + [zero] * 8)

    for blk in range(NBLK):
      row0 = w * RPW + blk * BROWS
      pltpu.sync_copy(srcm.at[pl.ds(row0, BROWS)], sbuf)
      pltpu.sync_copy(dstm.at[pl.ds(row0, BROWS)], dbuf)
      chs = [pltpu.async_copy(ones_v, cnt_acc.at[dbuf.at[kk]], sem_c,
                              add=True) for kk in range(BROWS)]

      def grp(g, st):
        wps = list(st[:8])
        fls = list(st[8:])
        grow = g // (SEG // L)
        goff = lax.rem(g, SEG // L) * L
        s16 = sbuf[grow, pl.ds(goff, L)]
        d16 = dbuf[grow, pl.ds(goff, L)]
        for bk in range(R):
          lo = bk * N
          m = (d16 >= lo) & (d16 < lo + N)
          mi = jnp.where(m, jnp.ones((L,), jnp.int32),
                         jnp.zeros((L,), jnp.int32))
          cs = plsc.cumsum(mi)
          n = jnp.sum(mi)
          base = jnp.full((L,), wps[bk] + RING - 1, jnp.int32)
          pos = lax.rem(base + cs, RING)
          plsc.store_scatter(ring_s.at[bk], [pos], s16, mask=m)
          plsc.store_scatter(ring_r.at[bk], [pos], d16 - lo, mask=m)
          wp2 = wps[bk] + n
          do = (wp2 - fls[bk]) >= FSEG
          fl = fls[bk]

          @pl.when(do)
          def _(bk=bk, fl=fl):
            @pl.when(fl > 0)
            def _():
              pltpu.make_async_copy(
                  ring_s.at[bk, pl.ds(0, FSEG)],
                  lsrc.at[w, bk, pl.ds(0, FSEG)], sem_f[bk]).wait()
              pltpu.make_async_copy(
                  ring_r.at[bk, pl.ds(0, FSEG)],
                  lrow.at[w, bk, pl.ds(0, FSEG)], sem_f[bk]).wait()
            o = pl.multiple_of(lax.rem(fl, RING), FSEG)
            flm = pl.multiple_of(fl, FSEG)
            pltpu.async_copy(ring_s.at[bk, pl.ds(o, FSEG)],
                             lsrc.at[w, bk, pl.ds(flm, FSEG)], sem_f[bk])
            pltpu.async_copy(ring_r.at[bk, pl.ds(o, FSEG)],
                             lrow.at[w, bk, pl.ds(flm, FSEG)], sem_f[bk])

          wps[bk] = wp2
          fls[bk] = jnp.where(do, fl + FSEG, fl)
        return tuple(wps + fls)

      state = lax.fori_loop(0, NGRP, grp, state)
      for h in chs:
        h.wait()

    nsv = jnp.zeros((L,), jnp.int32)
    for bk in range(R):
      wp = state[bk]
      fl = state[8 + bk]
      p = lax.rem(FSEG - lax.rem(wp, FSEG), FSEG)
      pv = jnp.full((L,), p, jnp.int32)
      wpv = jnp.full((L,), wp, jnp.int32)
      for t in range(FSEG // L):
        m = (t * L + it16) < pv
        pos = lax.rem(wpv + t * L + it16, RING)
        plsc.store_scatter(ring_s.at[bk], [pos],
                           jnp.zeros((L,), jnp.int32), mask=m)
        plsc.store_scatter(ring_r.at[bk], [pos],
                           DUMP_N + t * L + it16, mask=m)
      wp2 = wp + p

      @pl.when(fl > 0)
      def _(bk=bk):
        pltpu.make_async_copy(ring_s.at[bk, pl.ds(0, FSEG)],
                              lsrc.at[w, bk, pl.ds(0, FSEG)],
                              sem_f[bk]).wait()
        pltpu.make_async_copy(ring_r.at[bk, pl.ds(0, FSEG)],
                              lrow.at[w, bk, pl.ds(0, FSEG)],
                              sem_f[bk]).wait()

      @pl.when(wp2 > fl)
      def _(bk=bk, fl=fl):
        o = pl.multiple_of(lax.rem(fl, RING), FSEG)
        flm = pl.multiple_of(fl, FSEG)
        pltpu.sync_copy(ring_s.at[bk, pl.ds(o, FSEG)],
                        lsrc.at[w, bk, pl.ds(flm, FSEG)])
        pltpu.sync_copy(ring_r.at[bk, pl.ds(o, FSEG)],
                        lrow.at[w, bk, pl.ds(flm, FSEG)])
      nsv = jnp.where(it16 == bk,
                      jnp.full((L,), wp2 // FSEG, jnp.int32), nsv)

    nsvb[...] = nsv
    pltpu.sync_copy(nsvb, nseg_o.at[w])
    plsc.subcore_barrier()
    wb0 = sid * (RN // NTILES)

    @pl.when(cid == 0)
    def _():
      pltpu.sync_copy(cnt_acc.at[pl.ds(wb0, RN // NTILES)],
                      cnt_a.at[pl.ds(wb0, RN // NTILES)])

    @pl.when(cid == 1)
    def _():
      pltpu.sync_copy(cnt_acc.at[pl.ds(wb0, RN // NTILES)],
                      cnt_b.at[pl.ds(wb0, RN // NTILES)])

  out_type = (
      jax.ShapeDtypeStruct((NW, R, CAP), jnp.int32),
      jax.ShapeDtypeStruct((NW, R, CAP), jnp.int32),
      jax.ShapeDtypeStruct((NW, L), jnp.int32),
      jax.ShapeDtypeStruct((RN,), jnp.float32),
      jax.ShapeDtypeStruct((RN,), jnp.float32),
  )
  scratch = (
      pltpu.VMEM_SHARED((CNT_ROWS,), jnp.float32),
      pltpu.VMEM((BROWS, SEG), jnp.int32),
      pltpu.VMEM((BROWS, SEG), jnp.int32),
      pltpu.VMEM((R, RING), jnp.int32),
      pltpu.VMEM((R, RING), jnp.int32),
      pltpu.VMEM((L,), jnp.int32),
      pltpu.VMEM((SEG,), jnp.float32),
      pltpu.SemaphoreType.DMA,
  ) + (pltpu.SemaphoreType.DMA,) * R
  return pl.kernel(body, out_type=out_type, mesh=_mesh,
                   scratch_types=scratch, compiler_params=_params)


def _make_pass(n_half):
  def body(tab, lsrc, lrow, nseg, s_out, acc, isrc, irow, upd, zbuf, nsegv,
           sem_g):
    cid = lax.axis_index("c")
    sid = lax.axis_index("s")

    def fill_zbuf(i, c):
      for jj in range(D_IN // L):
        zbuf[i, pl.ds(jj * L, L)] = jnp.zeros((L,), jnp.float32)
      return c
    lax.fori_loop(0, ZR, fill_zbuf, 0)
    pltpu.sync_copy(nseg.at[pl.ds(2 * sid, 2)], nsegv)

    for r_l in range(R // 2):
      r = cid * (R // 2) + r_l
      for hf in range(n_half):
        z0 = sid * (ACC_N // NTILES)

        def zero_blk(k, carry):
          pltpu.sync_copy(zbuf, acc.at[pl.ds(z0 + k * ZR, ZR)])
          return carry
        lax.fori_loop(0, (ACC_N // NTILES) // ZR, zero_blk, 0)
        plsc.subcore_barrier()

        for q in range(2):
          reg = 2 * sid + q
          nv = nsegv[q, :]
          it16 = lax.iota(jnp.int32, L)
          rv = jnp.full((L,), r, jnp.int32)
          ns = jnp.sum(jnp.where(it16 == rv, nv, jnp.zeros_like(nv)))

          def seg_body(k, carry):
            pltpu.sync_copy(lsrc.at[reg, r, pl.ds(k * FSEG, FSEG)], isrc)
            pltpu.sync_copy(lrow.at[reg, r, pl.ds(k * FSEG, FSEG)], irow)
            if n_half == 1:
              pltpu.async_copy(tab.at[isrc], upd, sem_g).wait()
            else:
              pltpu.async_copy(tab.at[hf].at[isrc], upd, sem_g).wait()
            pltpu.sync_copy(upd, acc.at[irow], add=True)
            return carry
          lax.fori_loop(0, ns, seg_body, 0)

        plsc.subcore_barrier()
        wb0 = sid * WB
        if n_half == 1:
          pltpu.sync_copy(acc.at[pl.ds(wb0, WB)],
                          s_out.at[r, pl.ds(wb0, WB), :])
        else:
          pltpu.sync_copy(acc.at[pl.ds(wb0, WB)],
                          s_out.at[r, pl.ds(wb0, WB), hf, :])
        plsc.subcore_barrier()

  if n_half == 1:
    oshape = (R, N, D_IN)
  else:
    oshape = (R, N, n_half, D_IN)
  scratch = (
      pltpu.VMEM_SHARED((ACC_N, D_IN), jnp.float32),
      pltpu.VMEM((FSEG,), jnp.int32),
      pltpu.VMEM((FSEG,), jnp.int32),
      pltpu.VMEM((FSEG, D_IN), jnp.float32),
      pltpu.VMEM((ZR, D_IN), jnp.float32),
      pltpu.VMEM((2, L), jnp.int32),
      pltpu.SemaphoreType.DMA,
  )
  return pl.kernel(body,
                   out_type=jax.ShapeDtypeStruct(oshape, jnp.float32),
                   mesh=_mesh, scratch_types=scratch,
                   compiler_params=_params)


def _tc_layer(x, s, ca, cb, w_root, w_rel, b, relu):
  n, d_in = x.shape
  d_out = w_root.shape[1]
  mb = 1000
  g = n // mb

  def tc_body(x_ref, s_ref, ca_ref, cb_ref, wr_ref, wl_ref, b_ref, o_ref):
    inv = 1.0 / jnp.maximum(ca_ref[...] + cb_ref[...], 1.0)
    acc = jnp.dot(x_ref[...], wr_ref[...],
                  preferred_element_type=jnp.float32) + b_ref[...]
    for r in range(R):
      acc = acc + jnp.dot(s_ref[r] * inv[:, r:r + 1], wl_ref[r],
                          preferred_element_type=jnp.float32)
    o_ref[...] = jnp.maximum(acc, 0.0) if relu else acc

  return pl.pallas_call(
      tc_body,
      grid=(g,),
      in_specs=[
          pl.BlockSpec((mb, d_in), lambda i: (i, 0)),
          pl.BlockSpec((R, mb, d_in), lambda i: (0, i, 0)),
          pl.BlockSpec((mb, R), lambda i: (i, 0)),
          pl.BlockSpec((mb, R), lambda i: (i, 0)),
          pl.BlockSpec((d_in, d_out), lambda i: (0, 0)),
          pl.BlockSpec((R, d_in, d_out), lambda i: (0, 0, 0)),
          pl.BlockSpec((1, d_out), lambda i: (0, 0)),
      ],
      out_specs=pl.BlockSpec((mb, d_out), lambda i: (i, 0)),
      out_shape=jax.ShapeDtypeStruct((n, d_out), jnp.float32),
  )(x, s, ca, cb, w_root, w_rel, b.reshape(1, -1))


def kernel(x, edge_index, edge_type, W1_rel, W1_root, b1, W2_rel, W2_root,
           b2):
  src = edge_index[0].astype(jnp.int32)
  dst = edge_index[1].astype(jnp.int32)
  npad = E_PAD - E
  pad_src = jnp.zeros((npad,), jnp.int32)
  pad_dump = RN + jnp.arange(npad, dtype=jnp.int32) % (CNT_ROWS - RN)
  sidx = jnp.concatenate(
      [edge_type.astype(jnp.int32) * N + dst, pad_dump]).reshape(ROWS_E, SEG)
  srcm = jnp.concatenate([src, pad_src]).reshape(ROWS_E, SEG)
  zcnt = jnp.zeros((CNT_ROWS // NTILES,), jnp.float32)

  lsrc, lrow, nsg, cnt_a, cnt_b = _partition()(srcm, sidx, zcnt)
  ca = cnt_a.reshape(R, N).T
  cb = cnt_b.reshape(R, N).T

  s1 = _make_pass(1)(x, lsrc, lrow, nsg)
  h = _tc_layer(x, s1, ca, cb, W1_root, W1_rel, b1, relu=True)

  hH = h.reshape(N, 2, D_IN).transpose(1, 0, 2)
  s2h = _make_pass(2)(hH, lsrc, lrow, nsg)
  s2 = s2h.reshape(R, N, D_HID)

  return _tc_layer(h, s2, ca, cb, W2_root, W2_rel, b2, relu=False)


# speedup vs baseline: 2.1373x; 2.1373x over previous
import functools

import jax
import jax.numpy as jnp
from jax import lax
from jax.experimental import pallas as pl
from jax.experimental.pallas import tpu as pltpu
from jax.experimental.pallas import tpu_sc as plsc

N = 10000
E = 320000
R = 8
D_IN = 128
D_HID = 256
D_OUT = 128

L = 16
SEG = 128
NSEG = 10
BLK = SEG * NSEG
NTILES = 16
E_PAD = 327680
EPT = E_PAD // NTILES
NBLK = EPT // BLK
ROWS_E = E_PAD // SEG
RN = R * N
ACC_ROWS = 81920
APT = ACC_ROWS // NTILES
RPT = RN // NTILES
ZROWS = 128


def _make_scatter(c_total, with_counts):
  cp = c_total // 2
  mesh = plsc.VectorSubcoreMesh(core_axis_name="c", subcore_axis_name="s")

  def body(*refs):
    if with_counts:
      (xT, srcm, dstm, zcnt, s4, cnt_o, acc, cnt_acc, tab, idx_s, idx_d,
       upd, ones_v, zbuf, sem_i, sem_g, sem_s) = refs
    else:
      (xT, srcm, dstm, s4, acc, tab, idx_s, idx_d, upd, zbuf,
       sem_i, sem_g, sem_s) = refs
    cid = lax.axis_index("c")
    sid = lax.axis_index("s")
    t5 = sid * RPT
    z0 = sid * APT

    def fill_zbuf(i, c):
      zbuf[i, :] = jnp.zeros((L,), jnp.float32)
      return c
    lax.fori_loop(0, ZROWS, fill_zbuf, 0)

    if with_counts:
      def fill_ones(i, c):
        ones_v[pl.ds(i * L, L)] = jnp.ones((L,), jnp.float32)
        return c
      lax.fori_loop(0, SEG // L, fill_ones, 0)

      @pl.when(cid == 0)
      def _():
        pltpu.sync_copy(zcnt, cnt_acc.at[pl.ds(sid * (ACC_ROWS // NTILES),
                                               ACC_ROWS // NTILES)])

    for c_l in range(cp):
      c = cid * cp + c_l
      pltpu.sync_copy(xT.at[c, pl.ds(sid * (N // NTILES), N // NTILES)],
                      tab.at[pl.ds(sid * (N // NTILES), N // NTILES)])
      def zero_blk(k, carry):
        pltpu.sync_copy(zbuf, acc.at[pl.ds(z0 + k * ZROWS, ZROWS)])
        return carry
      lax.fori_loop(0, APT // ZROWS, zero_blk, 0)
      plsc.subcore_barrier()

      def blk_body(b, carry):
        row0 = sid * (EPT // SEG) + b * NSEG
        i1 = pltpu.async_copy(srcm.at[pl.ds(row0, NSEG)], idx_s, sem_i)
        i2 = pltpu.async_copy(dstm.at[pl.ds(row0, NSEG)], idx_d, sem_i)
        i1.wait()
        i2.wait()
        gs = [pltpu.async_copy(tab.at[idx_s.at[j]],
                               upd.at[pl.ds(j * SEG, SEG)], sem_g)
              for j in range(NSEG)]
        h = NSEG // 2
        for j in range(h):
          gs[j].wait()
        ss = [pltpu.async_copy(upd.at[pl.ds(j * SEG, SEG)],
                               acc.at[idx_d.at[j]], sem_s, add=True)
              for j in range(h)]
        for j in range(h, NSEG):
          gs[j].wait()
        ss += [pltpu.async_copy(upd.at[pl.ds(j * SEG, SEG)],
                                acc.at[idx_d.at[j]], sem_s, add=True)
               for j in range(h, NSEG)]
        if with_counts and c_l == 0:
          @pl.when(cid == 0)
          def _():
            cs = [pltpu.async_copy(ones_v, cnt_acc.at[idx_d.at[j]], sem_s,
                                   add=True)
                  for j in range(NSEG)]
            for hh in cs:
              hh.wait()
        for hh in ss:
          hh.wait()
        return carry

      lax.fori_loop(0, NBLK, blk_body, 0)
      plsc.subcore_barrier()
      pltpu.sync_copy(acc.at[pl.ds(t5, RPT)], s4.at[pl.ds(t5, RPT), c, :])
      if with_counts and c_l == 0:
        @pl.when(cid == 0)
        def _():
          pltpu.sync_copy(cnt_acc.at[pl.ds(t5, RPT)],
                          cnt_o.at[pl.ds(t5, RPT)])
      plsc.subcore_barrier()

  out_type = [jax.ShapeDtypeStruct((RN, c_total, L), jnp.float32)]
  scratch = [
      pltpu.VMEM_SHARED((ACC_ROWS, L), jnp.float32),
  ]
  if with_counts:
    out_type.append(jax.ShapeDtypeStruct((RN,), jnp.float32))
    scratch.append(pltpu.VMEM_SHARED((ACC_ROWS,), jnp.float32))
  scratch += [
      pltpu.VMEM_SHARED((N, L), jnp.float32),
      pltpu.VMEM((NSEG, SEG), jnp.int32),
      pltpu.VMEM((NSEG, SEG), jnp.int32),
      pltpu.VMEM((BLK, L), jnp.float32),
  ]
  if with_counts:
    scratch.append(pltpu.VMEM((SEG,), jnp.float32))
  scratch.append(pltpu.VMEM((ZROWS, L), jnp.float32))
  scratch += [pltpu.SemaphoreType.DMA, pltpu.SemaphoreType.DMA,
              pltpu.SemaphoreType.DMA]

  return pl.kernel(
      body, out_type=tuple(out_type), mesh=mesh,
      scratch_types=tuple(scratch),
      compiler_params=pltpu.CompilerParams(use_tc_tiling_on_sc=False))


def _tc_layer(x, s, cnt_t, w_root, w_rel, b, relu):
  n, d_in = x.shape
  d_out = w_root.shape[1]
  mb = 1000
  g = n // mb

  def tc_body(x_ref, s_ref, c_ref, wr_ref, wl_ref, b_ref, o_ref):
    inv = 1.0 / jnp.maximum(c_ref[...], 1.0)
    acc = jnp.dot(x_ref[...], wr_ref[...],
                  preferred_element_type=jnp.float32) + b_ref[...]
    for r in range(R):
      acc = acc + jnp.dot(s_ref[r] * inv[:, r:r + 1], wl_ref[r],
                          preferred_element_type=jnp.float32)
    o_ref[...] = jnp.maximum(acc, 0.0) if relu else acc

  return pl.pallas_call(
      tc_body,
      grid=(g,),
      in_specs=[
          pl.BlockSpec((mb, d_in), lambda i: (i, 0)),
          pl.BlockSpec((R, mb, d_in), lambda i: (0, i, 0)),
          pl.BlockSpec((mb, R), lambda i: (i, 0)),
          pl.BlockSpec((d_in, d_out), lambda i: (0, 0)),
          pl.BlockSpec((R, d_in, d_out), lambda i: (0, 0, 0)),
          pl.BlockSpec((1, d_out), lambda i: (0, 0)),
      ],
      out_specs=pl.BlockSpec((mb, d_out), lambda i: (i, 0)),
      out_shape=jax.ShapeDtypeStruct((n, d_out), jnp.float32),
  )(x, s, cnt_t, w_root, w_rel, b.reshape(1, -1))


def kernel(x, edge_index, edge_type, W1_rel, W1_root, b1, W2_rel, W2_root,
           b2):
  src = edge_index[0].astype(jnp.int32)
  dst = edge_index[1].astype(jnp.int32)
  npad = E_PAD - E
  pad_src = jnp.arange(npad, dtype=jnp.int32) % N
  pad_dump = RN + jnp.arange(npad, dtype=jnp.int32) % (ACC_ROWS - RN)
  sidx = jnp.concatenate(
      [edge_type.astype(jnp.int32) * N + dst, pad_dump]).reshape(ROWS_E, SEG)
  srcm = jnp.concatenate([src, pad_src]).reshape(ROWS_E, SEG)

  zcnt = jnp.zeros((ACC_ROWS // NTILES,), jnp.float32)
  xT = x.reshape(N, D_IN // L, L).transpose(1, 0, 2)
  s4, cnt = _make_scatter(D_IN // L, True)(xT, srcm, sidx, zcnt)
  s1 = s4.reshape(R, N, D_IN)
  cnt_t = cnt.reshape(R, N).T

  h = _tc_layer(x, s1, cnt_t, W1_root, W1_rel, b1, relu=True)

  hT = h.reshape(N, D_HID // L, L).transpose(1, 0, 2)
  s4b, = _make_scatter(D_HID // L, False)(hT, srcm, sidx)
  s2 = s4b.reshape(R, N, D_HID)

  return _tc_layer(h, s2, cnt_t, W2_root, W2_rel, b2, relu=False)
